# Initial kernel scaffold; baseline (speedup 1.0000x reference)
import jax
import jax.numpy as jnp
from jax.experimental import pallas as pl


def kernel(m, coords_pix, kernel):
    raise NotImplementedError("write your pallas kernel here")


# speedup vs baseline: 2.4544x; 2.4544x over previous
import functools

import jax
import jax.numpy as jnp
import numpy as np
from jax import lax
from jax.experimental import pallas as pl
from jax.experimental.pallas import tpu as pltpu
from jax.experimental.pallas import tpu_sc as plsc

_B, _H, _W = 256, 512, 512
_K = 33
_KH = 16
_NW = 32
_BPW = _B // _NW
_T = 1104
_NC = _T // 16

_t = np.arange(_T)
_KTAB = np.where(_t < _K * _K, _t // _K, 0).astype(np.int32)
_JM16 = np.where(_t < _K * _K, _t % _K - _KH, 0).astype(np.int32)


def _body(img_hbm, coords_hbm, kw_hbm, ktab_hbm, jm16_hbm, out_hbm,
          coords_v, idx_v, rows_v, kw_v, ktab_v, jm16_v, res_v, sem):
    wid = lax.axis_index("c") * 16 + lax.axis_index("s")
    base = wid * _BPW
    iota = lax.iota(jnp.int32, 16)

    pltpu.sync_copy(kw_hbm, kw_v)
    pltpu.sync_copy(ktab_hbm, ktab_v)
    pltpu.sync_copy(jm16_hbm, jm16_v)
    pltpu.sync_copy(coords_hbm.at[pl.ds(base * 2, 16)], coords_v)
    cv = coords_v[...]
    res = lax.full((16,), 0.0, jnp.float32)

    for i in range(_BPW):
        x0 = cv[2 * i]
        y0 = cv[2 * i + 1]
        rowbase = (base + i) * _W
        for off in (0, 16, 17):
            tv = iota + off
            gidx = ((y0 - _KH + tv) & (_H - 1)) + rowbase
            idx_v[pl.ds(off, 16)] = gidx
        pltpu.async_copy(img_hbm.at[idx_v], rows_v, sem).wait()

        acc = lax.full((16,), 0.0, jnp.float32)
        for c in range(_NC):
            tk = ktab_v[pl.ds(c * 16, 16)]
            tj = jm16_v[pl.ds(c * 16, 16)]
            w = kw_v[pl.ds(c * 16, 16)]
            col = (x0 + tj) & (_W - 1)
            acc = acc + plsc.load_gather(rows_v, [tk, col]) * w
        tot = jnp.sum(acc)
        res = jnp.where(iota == i, jnp.full((16,), tot, jnp.float32), res)

    res_v[...] = res
    pltpu.sync_copy(res_v, out_hbm.at[wid])


@jax.jit
def _run(img, coords_flat, kw, ktab, jm16):
    mesh = plsc.VectorSubcoreMesh(core_axis_name="c", subcore_axis_name="s")
    f = pl.kernel(
        _body,
        out_type=jax.ShapeDtypeStruct((_NW, 16), jnp.float32),
        mesh=mesh,
        compiler_params=pltpu.CompilerParams(needs_layout_passes=False,
                                             use_tc_tiling_on_sc=False),
        scratch_types=[
            pltpu.VMEM((16,), jnp.int32),
            pltpu.VMEM((_K,), jnp.int32),
            pltpu.VMEM((_K, _W), jnp.float32),
            pltpu.VMEM((_T,), jnp.float32),
            pltpu.VMEM((_T,), jnp.int32),
            pltpu.VMEM((_T,), jnp.int32),
            pltpu.VMEM((16,), jnp.float32),
            pltpu.SemaphoreType.DMA,
        ],
    )
    return f(img, coords_flat, kw, ktab, jm16)


def kernel(m, coords_pix, kernel):
    img = m.reshape(_B * _H, _W)
    coords_flat = coords_pix.astype(jnp.int32).reshape(-1)
    kw = jnp.concatenate(
        [kernel.reshape(-1), jnp.zeros((_T - _K * _K,), jnp.float32)])
    buf = _run(img, coords_flat, kw, jnp.asarray(_KTAB), jnp.asarray(_JM16))
    return buf[:, :_BPW].reshape(_B, 1)


# speedup vs baseline: 12.8285x; 5.2267x over previous
import functools

import jax
import jax.numpy as jnp
import numpy as np
from jax import lax
from jax.experimental import pallas as pl
from jax.experimental.pallas import tpu as pltpu
from jax.experimental.pallas import tpu_sc as plsc

_B, _H, _W = 256, 512, 512
_K = 33
_KP = 40
_KH = 16
_NW = 32
_BPW = _B // _NW
_T = 1104
_NC = _T // 16

_t = np.arange(_T)
_KTAB = np.where(_t < _K * _K, _t // _K, 0).astype(np.int32)
_JM16 = np.where(_t < _K * _K, _t % _K - _KH, 0).astype(np.int32)


def _body(img_hbm, coords_hbm, kw_hbm, ktab_hbm, jm16_hbm, out_hbm,
          coords_v, idx_v, rows_v, kw_v, ktab_v, jm16_v, res_v, sem):
    wid = lax.axis_index("c") * 16 + lax.axis_index("s")
    base = wid * _BPW
    iota = lax.iota(jnp.int32, 16)

    pltpu.sync_copy(kw_hbm, kw_v)
    pltpu.sync_copy(ktab_hbm, ktab_v)
    pltpu.sync_copy(jm16_hbm, jm16_v)
    pltpu.sync_copy(coords_hbm.at[pl.ds(base * 2, 16)], coords_v)
    cv = coords_v[...]
    res = lax.full((16,), 0.0, jnp.float32)

    for i in range(_BPW):
        x0 = cv[2 * i]
        y0 = cv[2 * i + 1]
        rowbase = (base + i) * _W
        for off in (0, 16, 24):
            tv = iota + off
            gidx = ((y0 - _KH + tv) & (_H - 1)) + rowbase
            idx_v[pl.ds(off, 16)] = gidx
        pltpu.async_copy(img_hbm.at[idx_v], rows_v, sem).wait()

        acc = lax.full((16,), 0.0, jnp.float32)
        for c in range(_NC):
            tk = ktab_v[pl.ds(c * 16, 16)]
            tj = jm16_v[pl.ds(c * 16, 16)]
            w = kw_v[pl.ds(c * 16, 16)]
            col = (x0 + tj) & (_W - 1)
            acc = acc + plsc.load_gather(rows_v, [tk, col]) * w
        tot = jnp.sum(acc)
        res = jnp.where(iota == i, jnp.full((16,), tot, jnp.float32), res)

    res_v[...] = res
    pltpu.sync_copy(res_v, out_hbm.at[wid])


@jax.jit
def _run(img, coords_flat, kw, ktab, jm16):
    mesh = plsc.VectorSubcoreMesh(core_axis_name="c", subcore_axis_name="s")
    f = pl.kernel(
        _body,
        out_type=jax.ShapeDtypeStruct((_NW, 16), jnp.float32),
        mesh=mesh,
        compiler_params=pltpu.CompilerParams(needs_layout_passes=False,
                                             use_tc_tiling_on_sc=True),
        scratch_types=[
            pltpu.VMEM((16,), jnp.int32),
            pltpu.VMEM((_KP,), jnp.int32),
            pltpu.VMEM((_KP, _W), jnp.float32),
            pltpu.VMEM((_T,), jnp.float32),
            pltpu.VMEM((_T,), jnp.int32),
            pltpu.VMEM((_T,), jnp.int32),
            pltpu.VMEM((16,), jnp.float32),
            pltpu.SemaphoreType.DMA,
        ],
    )
    return f(img, coords_flat, kw, ktab, jm16)


def kernel(m, coords_pix, kernel):
    img = m.reshape(_B * _H, _W)
    coords_flat = coords_pix.astype(jnp.int32).reshape(-1)
    kw = jnp.concatenate(
        [kernel.reshape(-1), jnp.zeros((_T - _K * _K,), jnp.float32)])
    buf = _run(img, coords_flat, kw, jnp.asarray(_KTAB), jnp.asarray(_JM16))
    return buf[:, :_BPW].reshape(_B, 1)


# speedup vs baseline: 13.7246x; 1.0698x over previous
import functools

import jax
import jax.numpy as jnp
import numpy as np
from jax import lax
from jax.experimental import pallas as pl
from jax.experimental.pallas import tpu as pltpu
from jax.experimental.pallas import tpu_sc as plsc

_B, _H, _W = 256, 512, 512
_K = 33
_KP = 40
_KH = 16
_NW = 32
_BPW = _B // _NW
_T = 1104
_NC = _T // 16

_t = np.arange(_T)
_KTAB = np.where(_t < _K * _K, _t // _K, 0).astype(np.int32)
_JM16 = np.where(_t < _K * _K, _t % _K - _KH, 0).astype(np.int32)


def _body(img_hbm, coords_hbm, kw_hbm, ktab_hbm, jm16_hbm, out_hbm,
          coords_v, idx0_v, idx1_v, rows0_v, rows1_v, kw_v, ktab_v, jm16_v,
          res_v, sem0, sem1):
    wid = lax.axis_index("c") * 16 + lax.axis_index("s")
    base = wid * _BPW
    iota = lax.iota(jnp.int32, 16)
    idx_bufs = (idx0_v, idx1_v)
    row_bufs = (rows0_v, rows1_v)
    sems = (sem0, sem1)

    pltpu.sync_copy(kw_hbm, kw_v)
    pltpu.sync_copy(ktab_hbm, ktab_v)
    pltpu.sync_copy(jm16_hbm, jm16_v)
    pltpu.sync_copy(coords_hbm.at[pl.ds(base * 2, 16)], coords_v)
    cv = coords_v[...]
    res = lax.full((16,), 0.0, jnp.float32)

    def start_gather(i):
        y0 = cv[2 * i + 1]
        rowbase = (base + i) * _W
        idx_v = idx_bufs[i % 2]
        for off in (0, 16, 24):
            tv = iota + off
            idx_v[pl.ds(off, 16)] = ((y0 - _KH + tv) & (_H - 1)) + rowbase
        return pltpu.async_copy(img_hbm.at[idx_v], row_bufs[i % 2],
                                sems[i % 2])

    pending = start_gather(0)
    for i in range(_BPW):
        if i + 1 < _BPW:
            nxt = start_gather(i + 1)
        pending.wait()
        x0 = cv[2 * i]
        rows_v = row_bufs[i % 2]
        acc = lax.full((16,), 0.0, jnp.float32)
        for c in range(_NC):
            tk = ktab_v[pl.ds(c * 16, 16)]
            tj = jm16_v[pl.ds(c * 16, 16)]
            w = kw_v[pl.ds(c * 16, 16)]
            col = (x0 + tj) & (_W - 1)
            acc = acc + plsc.load_gather(rows_v, [tk, col]) * w
        tot = jnp.sum(acc)
        res = jnp.where(iota == i, jnp.full((16,), tot, jnp.float32), res)
        if i + 1 < _BPW:
            pending = nxt

    res_v[...] = res
    pltpu.sync_copy(res_v, out_hbm.at[wid])


@jax.jit
def _run(img, coords_flat, kw, ktab, jm16):
    mesh = plsc.VectorSubcoreMesh(core_axis_name="c", subcore_axis_name="s")
    f = pl.kernel(
        _body,
        out_type=jax.ShapeDtypeStruct((_NW, 16), jnp.float32),
        mesh=mesh,
        compiler_params=pltpu.CompilerParams(needs_layout_passes=False,
                                             use_tc_tiling_on_sc=True),
        scratch_types=[
            pltpu.VMEM((16,), jnp.int32),
            pltpu.VMEM((_KP,), jnp.int32),
            pltpu.VMEM((_KP,), jnp.int32),
            pltpu.VMEM((_KP, _W), jnp.float32),
            pltpu.VMEM((_KP, _W), jnp.float32),
            pltpu.VMEM((_T,), jnp.float32),
            pltpu.VMEM((_T,), jnp.int32),
            pltpu.VMEM((_T,), jnp.int32),
            pltpu.VMEM((16,), jnp.float32),
            pltpu.SemaphoreType.DMA,
            pltpu.SemaphoreType.DMA,
        ],
    )
    return f(img, coords_flat, kw, ktab, jm16)


def kernel(m, coords_pix, kernel):
    img = m.reshape(_B * _H, _W)
    coords_flat = coords_pix.astype(jnp.int32).reshape(-1)
    kw = jnp.concatenate(
        [kernel.reshape(-1), jnp.zeros((_T - _K * _K,), jnp.float32)])
    buf = _run(img, coords_flat, kw, jnp.asarray(_KTAB), jnp.asarray(_JM16))
    return buf[:, :_BPW].reshape(_B, 1)


# speedup vs baseline: 13.7488x; 1.0018x over previous
import functools

import jax
import jax.numpy as jnp
import numpy as np
from jax import lax
from jax.experimental import pallas as pl
from jax.experimental.pallas import tpu as pltpu
from jax.experimental.pallas import tpu_sc as plsc

_B, _H, _W = 256, 512, 512
_K = 33
_KP = 40
_KH = 16
_NW = 32
_BPW = _B // _NW
_T = _K * _K
_NC = 68

_t = np.arange(_T)
_KTAB = (_t // _K).astype(np.int32)
_JM16 = (_t % _K - _KH).astype(np.int32)


def _body(img_hbm, coords_hbm, kw_hbm, ktab_hbm, jm16_hbm, out_hbm,
          coords_v, idx0_v, idx1_v, rows0_v, rows1_v, kw_v, ktab_v, jm16_v,
          res_v, sem0, sem1):
    wid = lax.axis_index("c") * 16 + lax.axis_index("s")
    base = wid * _BPW
    iota = lax.iota(jnp.int32, 16)
    idx_bufs = (idx0_v, idx1_v)
    row_bufs = (rows0_v, rows1_v)
    sems = (sem0, sem1)

    pltpu.sync_copy(kw_hbm, kw_v)
    pltpu.sync_copy(ktab_hbm, ktab_v)
    pltpu.sync_copy(jm16_hbm, jm16_v)
    pltpu.sync_copy(coords_hbm.at[pl.ds(base * 2, 16)], coords_v)
    cv = coords_v[...]
    res = lax.full((16,), 0.0, jnp.float32)

    def start_gather(i):
        y0 = cv[2 * i + 1]
        rowbase = (base + i) * _W
        idx_v = idx_bufs[i % 2]
        for off in (0, 16, 24):
            tv = iota + off
            idx_v[pl.ds(off, 16)] = ((y0 - _KH + tv) & (_H - 1)) + rowbase
        return pltpu.async_copy(img_hbm.at[idx_v], row_bufs[i % 2],
                                sems[i % 2])

    pending = start_gather(0)
    for i in range(_BPW):
        if i + 1 < _BPW:
            nxt = start_gather(i + 1)
        pending.wait()
        x0 = cv[2 * i]
        rows_v = row_bufs[i % 2]
        acc = lax.full((16,), 0.0, jnp.float32)
        for c in range(_NC + 1):
            off = c * 16 if c < _NC else _T - 16
            tk = ktab_v[pl.ds(off, 16)]
            tj = jm16_v[pl.ds(off, 16)]
            w = kw_v[pl.ds(off, 16)]
            if c == _NC:
                w = w * (iota == 15).astype(jnp.float32)
            col = (x0 + tj) & (_W - 1)
            acc = acc + plsc.load_gather(rows_v, [tk, col]) * w
        tot = jnp.sum(acc)
        res = jnp.where(iota == i, jnp.full((16,), tot, jnp.float32), res)
        if i + 1 < _BPW:
            pending = nxt

    res_v[...] = res
    pltpu.sync_copy(res_v, out_hbm.at[wid])


@jax.jit
def _run(img, coords_flat, kw, ktab, jm16):
    mesh = plsc.VectorSubcoreMesh(core_axis_name="c", subcore_axis_name="s")
    f = pl.kernel(
        _body,
        out_type=jax.ShapeDtypeStruct((_NW, 16), jnp.float32),
        mesh=mesh,
        compiler_params=pltpu.CompilerParams(needs_layout_passes=False,
                                             use_tc_tiling_on_sc=True),
        scratch_types=[
            pltpu.VMEM((16,), jnp.int32),
            pltpu.VMEM((_KP,), jnp.int32),
            pltpu.VMEM((_KP,), jnp.int32),
            pltpu.VMEM((_KP, _W), jnp.float32),
            pltpu.VMEM((_KP, _W), jnp.float32),
            pltpu.VMEM((_T,), jnp.float32),
            pltpu.VMEM((_T,), jnp.int32),
            pltpu.VMEM((_T,), jnp.int32),
            pltpu.VMEM((16,), jnp.float32),
            pltpu.SemaphoreType.DMA,
            pltpu.SemaphoreType.DMA,
        ],
    )
    return f(img, coords_flat, kw, ktab, jm16)


def kernel(m, coords_pix, kernel):
    img = m.reshape(_B * _H, _W)
    coords_flat = coords_pix.reshape(-1).astype(jnp.int32)
    kw = kernel.reshape(-1)
    buf = _run(img, coords_flat, kw, jnp.asarray(_KTAB), jnp.asarray(_JM16))
    return buf[:, :_BPW].reshape(_B, 1)


# speedup vs baseline: 14.5525x; 1.0585x over previous
import jax
import jax.numpy as jnp
import numpy as np
from jax import lax
from jax.experimental import pallas as pl
from jax.experimental.pallas import tpu as pltpu
from jax.experimental.pallas import tpu_sc as plsc

_B, _H, _W = 256, 512, 512
_K = 33
_KH = 16
_NW = 32
_BPW = _B // _NW
_T = _K * _K
_NC = 66
_NBUF = 3

_t = np.arange(_T)
_KTAB = (_t // _K).astype(np.int32)
_JM16 = (_t % _K - _KH).astype(np.int32)


def _body(img_hbm, coords_hbm, kw_hbm, ktab_hbm, jm16_hbm, out_hbm,
          coords_v, idx0_v, idx1_v, idx2_v, rows0_v, rows1_v, rows2_v,
          last0_v, last1_v, last2_v, kw_v, ktab_v, jm16_v, res_v,
          sem0, sem1, sem2):
    wid = lax.axis_index("c") * 16 + lax.axis_index("s")
    base = wid * _BPW
    iota = lax.iota(jnp.int32, 16)
    idx_bufs = (idx0_v, idx1_v, idx2_v)
    row_bufs = (rows0_v, rows1_v, rows2_v)
    last_bufs = (last0_v, last1_v, last2_v)
    sems = (sem0, sem1, sem2)

    pltpu.sync_copy(coords_hbm.at[pl.ds(base * 2, 16)], coords_v)
    cv = coords_v[...]

    def start_gather(i):
        y0 = cv[2 * i + 1]
        rowbase = (base + i) * _W
        idx_v = idx_bufs[i % _NBUF]
        for off in (0, 16, 17):
            tv = iota + off
            idx_v[pl.ds(off, 16)] = ((y0 - _KH + tv) & (_H - 1)) + rowbase
        d0 = pltpu.async_copy(img_hbm.at[idx_v.at[pl.ds(0, 32)]],
                              row_bufs[i % _NBUF], sems[i % _NBUF])
        d1 = pltpu.async_copy(img_hbm.at[idx_v.at[pl.ds(32, 1)]],
                              last_bufs[i % _NBUF], sems[i % _NBUF])
        return d0, d1

    pend = {0: start_gather(0), 1: start_gather(1)}
    pltpu.sync_copy(kw_hbm, kw_v)
    pltpu.sync_copy(ktab_hbm, ktab_v)
    pltpu.sync_copy(jm16_hbm, jm16_v)
    res = lax.full((16,), 0.0, jnp.float32)

    for i in range(_BPW):
        if i + 2 < _BPW:
            pend[i + 2] = start_gather(i + 2)
        d0, d1 = pend.pop(i)
        d0.wait()
        d1.wait()
        x0 = cv[2 * i]
        rows_v = row_bufs[i % _NBUF]
        last_v = last_bufs[i % _NBUF]
        acc = lax.full((16,), 0.0, jnp.float32)
        for c in range(_NC + 3):
            off = c * 16 if c < _NC + 2 else _T - 16
            tj = jm16_v[pl.ds(off, 16)]
            w = kw_v[pl.ds(off, 16)]
            col = (x0 + tj) & (_W - 1)
            if c < _NC:
                tk = ktab_v[pl.ds(off, 16)]
                g = plsc.load_gather(rows_v, [tk, col])
            else:
                if c == _NC + 2:
                    w = w * (iota == 15).astype(jnp.float32)
                g = plsc.load_gather(last_v, [iota * 0, col])
            acc = acc + g * w
        tot = jnp.sum(acc)
        res = jnp.where(iota == i, jnp.full((16,), tot, jnp.float32), res)

    res_v[...] = res
    pltpu.sync_copy(res_v, out_hbm.at[wid])


@jax.jit
def _run(img, coords_flat, kw, ktab, jm16):
    mesh = plsc.VectorSubcoreMesh(core_axis_name="c", subcore_axis_name="s")
    f = pl.kernel(
        _body,
        out_type=jax.ShapeDtypeStruct((_NW, 16), jnp.float32),
        mesh=mesh,
        compiler_params=pltpu.CompilerParams(needs_layout_passes=False,
                                             use_tc_tiling_on_sc=True),
        scratch_types=[
            pltpu.VMEM((16,), jnp.int32),
            pltpu.VMEM((_K,), jnp.int32),
            pltpu.VMEM((_K,), jnp.int32),
            pltpu.VMEM((_K,), jnp.int32),
            pltpu.VMEM((32, _W), jnp.float32),
            pltpu.VMEM((32, _W), jnp.float32),
            pltpu.VMEM((32, _W), jnp.float32),
            pltpu.VMEM((1, _W), jnp.float32),
            pltpu.VMEM((1, _W), jnp.float32),
            pltpu.VMEM((1, _W), jnp.float32),
            pltpu.VMEM((_T,), jnp.float32),
            pltpu.VMEM((_T,), jnp.int32),
            pltpu.VMEM((_T,), jnp.int32),
            pltpu.VMEM((16,), jnp.float32),
            pltpu.SemaphoreType.DMA,
            pltpu.SemaphoreType.DMA,
            pltpu.SemaphoreType.DMA,
        ],
    )
    return f(img, coords_flat, kw, ktab, jm16)


def kernel(m, coords_pix, kernel):
    img = m.reshape(_B * _H, _W)
    coords_flat = coords_pix.reshape(-1).astype(jnp.int32)
    kw = kernel.reshape(-1)
    buf = _run(img, coords_flat, kw, jnp.asarray(_KTAB), jnp.asarray(_JM16))
    return buf[:, :_BPW].reshape(_B, 1)


# speedup vs baseline: 15.5206x; 1.0665x over previous
import jax
import jax.numpy as jnp
import numpy as np
from jax import lax
from jax.experimental import pallas as pl
from jax.experimental.pallas import tpu as pltpu
from jax.experimental.pallas import tpu_sc as plsc

_B, _H, _W = 256, 512, 512
_K = 33
_KH = 16
_NW = 32
_BPW = _B // _NW
_T = _K * _K
_NC = 66
_NBUF = 3

_t = np.arange(_T)
_KTAB = (_t // _K).astype(np.int32)
_JM16 = (_t % _K - _KH).astype(np.int32)


def _body(img_hbm, coords_hbm, kw_hbm, ktab_hbm, jm16_hbm, out_hbm,
          coords_v, idx0_v, idx1_v, idx2_v, rows0_v, rows1_v, rows2_v,
          last0_v, last1_v, last2_v, kw_v, ktab_v, jm16_v, res_v,
          sem0, sem1, sem2):
    wid = lax.axis_index("c") * 16 + lax.axis_index("s")
    base = wid * _BPW
    iota = lax.iota(jnp.int32, 16)
    idx_bufs = (idx0_v, idx1_v, idx2_v)
    row_bufs = (rows0_v, rows1_v, rows2_v)
    last_bufs = (last0_v, last1_v, last2_v)
    sems = (sem0, sem1, sem2)

    pltpu.sync_copy(coords_hbm.at[pl.ds(base * 2, 16)], coords_v)
    cv = coords_v[...]

    def start_gather(i):
        y0 = cv[2 * i + 1]
        rowbase = (base + i) * _W
        idx_v = idx_bufs[i % _NBUF]
        for off in (0, 16, 17):
            tv = iota + off
            idx_v[pl.ds(off, 16)] = ((y0 - _KH + tv) & (_H - 1)) + rowbase
        d0 = pltpu.async_copy(img_hbm.at[idx_v.at[pl.ds(0, 32)]],
                              row_bufs[i % _NBUF], sems[i % _NBUF])
        d1 = pltpu.async_copy(img_hbm.at[idx_v.at[pl.ds(32, 1)]],
                              last_bufs[i % _NBUF], sems[i % _NBUF])
        return d0, d1

    pend = {0: start_gather(0), 1: start_gather(1)}
    pltpu.sync_copy(kw_hbm, kw_v)
    pltpu.sync_copy(ktab_hbm, ktab_v)
    pltpu.sync_copy(jm16_hbm, jm16_v)
    res = lax.full((16,), 0.0, jnp.float32)

    for i in range(_BPW):
        if i + 2 < _BPW:
            pend[i + 2] = start_gather(i + 2)
        d0, d1 = pend.pop(i)
        d0.wait()
        d1.wait()
        x0 = cv[2 * i]
        rows_v = row_bufs[i % _NBUF]
        last_v = last_bufs[i % _NBUF]
        @pl.loop(0, _NC, init_carry=lax.full((16,), 0.0, jnp.float32))
        def acc_loop(c, acc):
            off = c * 16
            tj = jm16_v[pl.ds(off, 16)]
            w = kw_v[pl.ds(off, 16)]
            tk = ktab_v[pl.ds(off, 16)]
            col = (x0 + tj) & (_W - 1)
            return acc + plsc.load_gather(rows_v, [tk, col]) * w

        acc = acc_loop
        for c in range(_NC, _NC + 3):
            off = c * 16 if c < _NC + 2 else _T - 16
            tj = jm16_v[pl.ds(off, 16)]
            w = kw_v[pl.ds(off, 16)]
            if c == _NC + 2:
                w = w * (iota == 15).astype(jnp.float32)
            col = (x0 + tj) & (_W - 1)
            acc = acc + plsc.load_gather(last_v, [iota * 0, col]) * w
        tot = jnp.sum(acc)
        res = jnp.where(iota == i, jnp.full((16,), tot, jnp.float32), res)

    res_v[...] = res
    pltpu.sync_copy(res_v, out_hbm.at[wid])


@jax.jit
def _run(img, coords_flat, kw, ktab, jm16):
    mesh = plsc.VectorSubcoreMesh(core_axis_name="c", subcore_axis_name="s")
    f = pl.kernel(
        _body,
        out_type=jax.ShapeDtypeStruct((_NW, 16), jnp.float32),
        mesh=mesh,
        compiler_params=pltpu.CompilerParams(needs_layout_passes=False,
                                             use_tc_tiling_on_sc=True),
        scratch_types=[
            pltpu.VMEM((16,), jnp.int32),
            pltpu.VMEM((_K,), jnp.int32),
            pltpu.VMEM((_K,), jnp.int32),
            pltpu.VMEM((_K,), jnp.int32),
            pltpu.VMEM((32, _W), jnp.float32),
            pltpu.VMEM((32, _W), jnp.float32),
            pltpu.VMEM((32, _W), jnp.float32),
            pltpu.VMEM((1, _W), jnp.float32),
            pltpu.VMEM((1, _W), jnp.float32),
            pltpu.VMEM((1, _W), jnp.float32),
            pltpu.VMEM((_T,), jnp.float32),
            pltpu.VMEM((_T,), jnp.int32),
            pltpu.VMEM((_T,), jnp.int32),
            pltpu.VMEM((16,), jnp.float32),
            pltpu.SemaphoreType.DMA,
            pltpu.SemaphoreType.DMA,
            pltpu.SemaphoreType.DMA,
        ],
    )
    return f(img, coords_flat, kw, ktab, jm16)


def kernel(m, coords_pix, kernel):
    img = m.reshape(_B * _H, _W)
    coords_flat = coords_pix.reshape(-1).astype(jnp.int32)
    kw = kernel.reshape(-1)
    buf = _run(img, coords_flat, kw, jnp.asarray(_KTAB), jnp.asarray(_JM16))
    return buf[:, :_BPW].reshape(_B, 1)


# speedup vs baseline: 18.2633x; 1.1767x over previous
import jax
import jax.numpy as jnp
import numpy as np
from jax import lax
from jax.experimental import pallas as pl
from jax.experimental.pallas import tpu as pltpu
from jax.experimental.pallas import tpu_sc as plsc

_B, _H, _W = 256, 512, 512
_K = 33
_KH = 16
_NW = 32
_BPW = _B // _NW
_T = _K * _K
_NC = 66
_NBUF = 3


def _body(img_hbm, coords_hbm, kw_hbm, out_hbm,
          coords_v, idx0_v, idx1_v, idx2_v, rows0_v, rows1_v, rows2_v,
          last0_v, last1_v, last2_v, kw_v, res_v,
          sem0, sem1, sem2):
    wid = lax.axis_index("c") * 16 + lax.axis_index("s")
    base = wid * _BPW
    iota = lax.iota(jnp.int32, 16)
    idx_bufs = (idx0_v, idx1_v, idx2_v)
    row_bufs = (rows0_v, rows1_v, rows2_v)
    last_bufs = (last0_v, last1_v, last2_v)
    sems = (sem0, sem1, sem2)

    pltpu.sync_copy(coords_hbm.at[pl.ds(base * 2, 16)], coords_v)
    cv = coords_v[...]

    def start_gather(i):
        y0 = cv[2 * i + 1]
        rowbase = (base + i) * _W
        idx_v = idx_bufs[i % _NBUF]
        for off in (0, 16, 17):
            tv = iota + off
            idx_v[pl.ds(off, 16)] = ((y0 - _KH + tv) & (_H - 1)) + rowbase
        d0 = pltpu.async_copy(img_hbm.at[idx_v.at[pl.ds(0, 32)]],
                              row_bufs[i % _NBUF], sems[i % _NBUF])
        d1 = pltpu.async_copy(img_hbm.at[idx_v.at[pl.ds(32, 1)]],
                              last_bufs[i % _NBUF], sems[i % _NBUF])
        return d0, d1

    pend = {0: start_gather(0), 1: start_gather(1)}
    pltpu.sync_copy(kw_hbm, kw_v)
    res = lax.full((16,), 0.0, jnp.float32)

    for i in range(_BPW):
        if i + 2 < _BPW:
            pend[i + 2] = start_gather(i + 2)
        d0, d1 = pend.pop(i)
        d0.wait()
        d1.wait()
        x0 = cv[2 * i]
        rows_v = row_bufs[i % _NBUF]
        last_v = last_bufs[i % _NBUF]
        @pl.loop(0, _NC, init_carry=lax.full((16,), 0.0, jnp.float32))
        def acc_loop(c, acc):
            t = iota + c * 16
            w = kw_v[pl.ds(c * 16, 16)]
            tk = (t * 1986) >> 16
            col = (x0 + t - tk * 33 - _KH) & (_W - 1)
            return acc + plsc.load_gather(rows_v, [tk, col]) * w

        acc = acc_loop
        for c in range(_NC, _NC + 3):
            off = c * 16 if c < _NC + 2 else _T - 16
            w = kw_v[pl.ds(off, 16)]
            if c == _NC + 2:
                w = w * (iota == 15).astype(jnp.float32)
            col = (x0 + (iota + off) - 32 * 33 - _KH) & (_W - 1)
            acc = acc + plsc.load_gather(last_v, [iota * 0, col]) * w
        tot = jnp.sum(acc)
        res = jnp.where(iota == i, jnp.full((16,), tot, jnp.float32), res)

    res_v[...] = res
    pltpu.sync_copy(res_v, out_hbm.at[wid])


@jax.jit
def _run(img, coords_flat, kw):
    mesh = plsc.VectorSubcoreMesh(core_axis_name="c", subcore_axis_name="s")
    f = pl.kernel(
        _body,
        out_type=jax.ShapeDtypeStruct((_NW, 16), jnp.float32),
        mesh=mesh,
        compiler_params=pltpu.CompilerParams(needs_layout_passes=False,
                                             use_tc_tiling_on_sc=True),
        scratch_types=[
            pltpu.VMEM((16,), jnp.int32),
            pltpu.VMEM((_K,), jnp.int32),
            pltpu.VMEM((_K,), jnp.int32),
            pltpu.VMEM((_K,), jnp.int32),
            pltpu.VMEM((32, _W), jnp.float32),
            pltpu.VMEM((32, _W), jnp.float32),
            pltpu.VMEM((32, _W), jnp.float32),
            pltpu.VMEM((1, _W), jnp.float32),
            pltpu.VMEM((1, _W), jnp.float32),
            pltpu.VMEM((1, _W), jnp.float32),
            pltpu.VMEM((_T,), jnp.float32),
            pltpu.VMEM((16,), jnp.float32),
            pltpu.SemaphoreType.DMA,
            pltpu.SemaphoreType.DMA,
            pltpu.SemaphoreType.DMA,
        ],
    )
    return f(img, coords_flat, kw)


def kernel(m, coords_pix, kernel):
    img = m.reshape(_B * _H, _W)
    coords_flat = coords_pix.reshape(-1).astype(jnp.int32)
    kw = kernel.reshape(-1)
    buf = _run(img, coords_flat, kw)
    return buf[:, :_BPW].reshape(_B, 1)
